# baseline (speedup 1.0000x reference)
import functools
import math

import jax
import jax.numpy as jnp
from jax import lax
from jax.experimental import pallas as pl
from jax.experimental.pallas import tpu as pltpu
from jax.experimental.pallas import tpu_sc as plsc

D_MODEL = 64
SCALE = math.sqrt(D_MODEL)
NUM_CORES = 2
NUM_SUBCORES = 16
NUM_WORKERS = NUM_CORES * NUM_SUBCORES

SEQ = 200
BATCH = 4096
BTILES = BATCH // 128
UNITS = SEQ * BTILES
UNITS_PER_W = UNITS // NUM_WORKERS
LOOKUPS = 128
GROWS = 2 * LOOKUPS
IDX_PER_W = UNITS_PER_W * LOOKUPS
SPAD = 129


def _make_lookup():
    mesh = plsc.VectorSubcoreMesh(core_axis_name="c", subcore_axis_name="s")

    @functools.partial(
        pl.kernel,
        mesh=mesh,
        compiler_params=pltpu.CompilerParams(
            use_tc_tiling_on_sc=False, needs_layout_passes=False),
        out_type=jax.ShapeDtypeStruct((SEQ, 8, BTILES * 8, 128), jnp.float32),
        scratch_types=[
            pltpu.VMEM((IDX_PER_W,), jnp.int32),
            pltpu.VMEM((GROWS,), jnp.int32),
            pltpu.VMEM((GROWS,), jnp.int32),
            pltpu.VMEM((GROWS, 32), jnp.float32),
            pltpu.VMEM((GROWS, 32), jnp.float32),
            pltpu.VMEM((D_MODEL, SPAD), jnp.float32),
            pltpu.VMEM((D_MODEL, SPAD), jnp.float32),
            pltpu.SemaphoreType.DMA,
            pltpu.SemaphoreType.DMA,
            pltpu.SemaphoreType.DMA,
            pltpu.SemaphoreType.DMA,
        ],
    )
    def lookup(lut_hbm, idx_hbm, out_hbm, idx_all, i0, i1, g0, g1, st0, st1,
               sg0, sg1, sw0, sw1):
        wid = lax.axis_index("s") * NUM_CORES + lax.axis_index("c")
        base_idx = pl.multiple_of(wid * IDX_PER_W, 8)
        pltpu.sync_copy(idx_hbm.at[pl.ds(base_idx, IDX_PER_W)], idx_all)
        base_u = wid * UNITS_PER_W

        ibufs, gbufs, stages = (i0, i1), (g0, g1), (st0, st1)
        sgs, sws = (sg0, sg1), (sw0, sw1)
        iot = lax.iota(jnp.int32, 16)

        def start_gather(par, k):
            ib = ibufs[par]
            for g in range(8):
                v = idx_all[pl.ds(pl.multiple_of(k * LOOKUPS, 8) + g * 16, 16)]
                e = v * 2
                plsc.store_scatter(ib, [g * 32 + 2 * iot], e)
                plsc.store_scatter(ib, [g * 32 + 2 * iot + 1], e + 1)
            pltpu.async_copy(lut_hbm.at[ib], gbufs[par], sgs[par])

        def transpose_scale(par):
            gb, st = gbufs[par], stages[par]

            def rbody(i, biv):
                for dr in range(4):
                    r = i * 4 + dr
                    bv = biv + dr
                    for j in range(4):
                        vec = gb[2 * r + (j // 2), pl.ds((j % 2) * 16, 16)]
                        plsc.store_scatter(st, [j * 16 + iot, bv], vec * SCALE)
                return biv + 4

            lax.fori_loop(0, LOOKUPS // 4, rbody, jnp.zeros((16,), jnp.int32))

        def store_out(par, k):
            u = base_u + k
            s = u // BTILES
            bt = u % BTILES
            st = stages[par]
            for ft in range(8):
                pltpu.async_copy(
                    st.at[pl.ds(ft * 8, 8), pl.ds(0, 128)],
                    out_hbm.at[s, ft, pl.ds(bt * 8, 8)], sws[par])

        def wait_writes(par):
            for _ in range(8):
                pltpu.make_async_copy(
                    stages[par].at[pl.ds(0, 8), pl.ds(0, 128)],
                    out_hbm.at[0, 0, pl.ds(0, 8)], sws[par]).wait()

        def consume(par, k, pre_k):
            pltpu.make_async_copy(
                lut_hbm.at[ibufs[par]], gbufs[par], sgs[par]).wait()
            wait_writes(par)
            transpose_scale(par)
            store_out(par, k)
            start_gather(par, pre_k)

        start_gather(0, 0)
        start_gather(1, 1)
        store_out(0, 0)
        store_out(1, 1)

        def pair_body(o, carry):
            k0 = 2 * o
            consume(0, k0, jnp.minimum(k0 + 2, UNITS_PER_W - 2))
            consume(1, k0 + 1, jnp.minimum(k0 + 3, UNITS_PER_W - 1))
            return carry

        lax.fori_loop(0, UNITS_PER_W // 2, pair_body, 0)

        for par in (0, 1):
            wait_writes(par)
            pltpu.make_async_copy(
                lut_hbm.at[ibufs[par]], gbufs[par], sgs[par]).wait()

    return lookup


def kernel(x, lut):
    xt = x.T.reshape(-1)
    lut32 = lut.reshape(2 * lut.shape[0], 32)
    op = _make_lookup()(lut32, xt)
    return (op.reshape(SEQ, 8, BTILES, 8, 128)
            .transpose(2, 4, 0, 1, 3)
            .reshape(BATCH, SEQ, D_MODEL))


# speedup vs baseline: 1.0007x; 1.0007x over previous
import functools
import math

import jax
import jax.numpy as jnp
from jax import lax
from jax.experimental import pallas as pl
from jax.experimental.pallas import tpu as pltpu
from jax.experimental.pallas import tpu_sc as plsc

D_MODEL = 64
SCALE = math.sqrt(D_MODEL)
NUM_CORES = 2
NUM_SUBCORES = 16
NUM_WORKERS = NUM_CORES * NUM_SUBCORES

SEQ = 200
BATCH = 4096
BTILES = BATCH // 128
UNITS = SEQ * BTILES
UNITS_PER_W = UNITS // NUM_WORKERS
LOOKUPS = 128
GROWS = 2 * LOOKUPS
IDX_PER_W = UNITS_PER_W * LOOKUPS
SPAD = 129


def _make_lookup():
    mesh = plsc.VectorSubcoreMesh(core_axis_name="c", subcore_axis_name="s")

    @functools.partial(
        pl.kernel,
        mesh=mesh,
        compiler_params=pltpu.CompilerParams(
            use_tc_tiling_on_sc=False, needs_layout_passes=False),
        out_type=jax.ShapeDtypeStruct((SEQ, 8, BTILES * 8, 128), jnp.float32),
        scratch_types=[
            pltpu.VMEM((IDX_PER_W,), jnp.int32),
            pltpu.VMEM((GROWS,), jnp.int32),
            pltpu.VMEM((GROWS,), jnp.int32),
            pltpu.VMEM((GROWS, 32), jnp.float32),
            pltpu.VMEM((GROWS, 32), jnp.float32),
            pltpu.VMEM((D_MODEL, SPAD), jnp.float32),
            pltpu.VMEM((D_MODEL, SPAD), jnp.float32),
            pltpu.SemaphoreType.DMA,
            pltpu.SemaphoreType.DMA,
            pltpu.SemaphoreType.DMA,
            pltpu.SemaphoreType.DMA,
        ],
    )
    def lookup(lut_hbm, idx_hbm, out_hbm, idx_all, i0, i1, g0, g1, st0, st1,
               sg0, sg1, sw0, sw1):
        wid = lax.axis_index("s") * NUM_CORES + lax.axis_index("c")
        base_idx = pl.multiple_of(wid * IDX_PER_W, 8)
        pltpu.sync_copy(idx_hbm.at[pl.ds(base_idx, IDX_PER_W)], idx_all)
        base_u = wid * UNITS_PER_W

        ibufs, gbufs, stages = (i0, i1), (g0, g1), (st0, st1)
        sgs, sws = (sg0, sg1), (sw0, sw1)
        iot = lax.iota(jnp.int32, 16)

        def start_gather(par, k):
            ib = ibufs[par]
            for g in range(8):
                v = idx_all[pl.ds(pl.multiple_of(k * LOOKUPS, 8) + g * 16, 16)]
                e = v * 2
                plsc.store_scatter(ib, [g * 32 + 2 * iot], e)
                plsc.store_scatter(ib, [g * 32 + 2 * iot + 1], e + 1)
            pltpu.async_copy(lut_hbm.at[ib], gbufs[par], sgs[par])

        def transpose_scale(par):
            gb, st = gbufs[par], stages[par]

            def rbody(i, biv):
                for dr in range(4):
                    r = i * 4 + dr
                    bv = biv + dr
                    for j in range(4):
                        vec = gb[2 * r + (j // 2), pl.ds((j % 2) * 16, 16)]
                        plsc.store_scatter(st, [j * 16 + iot, bv], vec * SCALE)
                return biv + 4

            lax.fori_loop(0, LOOKUPS // 4, rbody, jnp.zeros((16,), jnp.int32))

        def store_out(par, k):
            u = base_u + k
            s = u // BTILES
            bt = u % BTILES
            st = stages[par]
            for ft in range(8):
                pltpu.async_copy(
                    st.at[pl.ds(ft * 8, 8), pl.ds(0, 128)],
                    out_hbm.at[s, ft, pl.ds(bt * 8, 8)], sws[par])

        def wait_writes(par):
            for _ in range(8):
                pltpu.make_async_copy(
                    stages[par].at[pl.ds(0, 8), pl.ds(0, 128)],
                    out_hbm.at[0, 0, pl.ds(0, 8)], sws[par]).wait()

        def consume(par, k, pre_k):
            pltpu.make_async_copy(
                lut_hbm.at[ibufs[par]], gbufs[par], sgs[par]).wait()
            wait_writes(par)
            transpose_scale(par)
            store_out(par, k)
            start_gather(par, pre_k)

        start_gather(0, 0)
        start_gather(1, 1)
        store_out(0, 0)
        store_out(1, 1)

        def pair_body(o, carry):
            k0 = 2 * o
            consume(0, k0, jnp.minimum(k0 + 2, UNITS_PER_W - 2))
            consume(1, k0 + 1, jnp.minimum(k0 + 3, UNITS_PER_W - 1))
            return carry

        lax.fori_loop(0, UNITS_PER_W // 2, pair_body, 0)

        for par in (0, 1):
            wait_writes(par)
            pltpu.make_async_copy(
                lut_hbm.at[ibufs[par]], gbufs[par], sgs[par]).wait()

    return lookup


TC_BLK_V = 2048


def _make_tc_transpose(vocab: int):
    steps = (vocab + TC_BLK_V - 1) // TC_BLK_V

    def body(in_ref, out_ref):
        xb = in_ref[...]
        xb = xb.reshape(D_MODEL, TC_BLK_V // 2, 2)
        yb = jnp.transpose(xb, (1, 2, 0))
        out_ref[...] = yb.reshape(TC_BLK_V // 2, 2 * D_MODEL)

    return pl.pallas_call(
        body,
        grid=(steps,),
        in_specs=[pl.BlockSpec((D_MODEL, TC_BLK_V), lambda i: (0, i))],
        out_specs=pl.BlockSpec((TC_BLK_V // 2, 2 * D_MODEL),
                               lambda i: (i, 0)),
        out_shape=jax.ShapeDtypeStruct((vocab // 2, 2 * D_MODEL),
                                       jnp.float32),
    )


def kernel(x, lut):
    xt = x.T.reshape(-1)
    lut128 = _make_tc_transpose(lut.shape[0])(lut.T)
    lut32 = lut128.reshape(2 * lut.shape[0], 32)
    op = _make_lookup()(lut32, xt)
    return (op.reshape(SEQ, 8, BTILES, 8, 128)
            .transpose(2, 4, 0, 1, 3)
            .reshape(BATCH, SEQ, D_MODEL))


# speedup vs baseline: 8.1917x; 8.1856x over previous
import functools
import math

import jax
import jax.numpy as jnp
from jax import lax
from jax.experimental import pallas as pl
from jax.experimental.pallas import tpu as pltpu
from jax.experimental.pallas import tpu_sc as plsc

D_MODEL = 64
SCALE = math.sqrt(D_MODEL)
NUM_CORES = 2
NUM_SUBCORES = 16
NUM_WORKERS = NUM_CORES * NUM_SUBCORES

SEQ = 200
BATCH = 4096
BTILES = BATCH // 128
UNITS = SEQ * (BTILES // 2)
UNITS_PER_W = UNITS // NUM_WORKERS
ROWS = 256
IDX_PER_W = UNITS_PER_W * ROWS
SPAD = 129


def _make_lookup():
    mesh = plsc.VectorSubcoreMesh(core_axis_name="c", subcore_axis_name="s")

    @functools.partial(
        pl.kernel,
        mesh=mesh,
        compiler_params=pltpu.CompilerParams(
            use_tc_tiling_on_sc=False, needs_layout_passes=False),
        out_type=jax.ShapeDtypeStruct((SEQ, 8, BTILES, 8, 128), jnp.float32),
        scratch_types=[
            pltpu.VMEM((IDX_PER_W,), jnp.int32),
            pltpu.VMEM((ROWS, D_MODEL), jnp.float32),
            pltpu.VMEM((ROWS, D_MODEL), jnp.float32),
            pltpu.VMEM((8, 2, 8, SPAD), jnp.float32),
            pltpu.VMEM((8, 2, 8, SPAD), jnp.float32),
            pltpu.SemaphoreType.DMA,
            pltpu.SemaphoreType.DMA,
            pltpu.SemaphoreType.DMA,
            pltpu.SemaphoreType.DMA,
        ],
    )
    def lookup(lut_hbm, idx_hbm, out_hbm, idx_all, g0, g1, st0, st1,
               sg0, sg1, sw0, sw1):
        wid = lax.axis_index("s") * NUM_CORES + lax.axis_index("c")
        base_idx = pl.multiple_of(wid * IDX_PER_W, 8)
        pltpu.sync_copy(idx_hbm.at[pl.ds(base_idx, IDX_PER_W)], idx_all)
        base_u = wid * UNITS_PER_W

        gbufs, stages = (g0, g1), (st0, st1)
        sgs, sws = (sg0, sg1), (sw0, sw1)
        iot = lax.iota(jnp.int32, 16)
        ftv = [(2 * j) + (iot >> 3) for j in range(4)]
        fiv = iot & 7
        btivs = [jnp.full((16,), bti, jnp.int32) for bti in range(2)]

        def idx_slice(k):
            return idx_all.at[pl.ds(pl.multiple_of(k * ROWS, 8), ROWS)]

        def start_gather(par, k):
            pltpu.async_copy(lut_hbm.at[idx_slice(k)], gbufs[par], sgs[par])

        def stage_view(par):
            return stages[par].at[:, :, :, pl.ds(0, 128)]

        def transpose_scale(par):
            gb, st = gbufs[par], stages[par]
            for bti in range(2):
                def rbody(i, biv, bti=bti):
                    for dr in range(4):
                        row = bti * 128 + i * 4 + dr
                        bv = biv + dr
                        for j in range(4):
                            vec = gb[row, pl.ds(j * 16, 16)] * SCALE
                            plsc.store_scatter(
                                st, [ftv[j], btivs[bti], fiv, bv], vec)
                    return biv + 4

                lax.fori_loop(0, 32, rbody, jnp.zeros((16,), jnp.int32))

        def out_slice(k):
            u = base_u + k
            s = u // (BTILES // 2)
            bp = u % (BTILES // 2)
            return out_hbm.at[s, :, pl.ds(2 * bp, 2)]

        def consume(par, k, pre_k):
            pltpu.make_async_copy(
                lut_hbm.at[idx_slice(k)], gbufs[par], sgs[par]).wait()
            pltpu.make_async_copy(
                stage_view(par), out_hbm.at[0, :, pl.ds(0, 2)],
                sws[par]).wait()
            transpose_scale(par)
            pltpu.async_copy(stage_view(par), out_slice(k), sws[par])
            start_gather(par, pre_k)

        start_gather(0, 0)
        start_gather(1, 1)
        pltpu.async_copy(stage_view(0), out_slice(0), sws[0])
        pltpu.async_copy(stage_view(1), out_slice(1), sws[1])

        def pair_body(o, carry):
            k0 = 2 * o
            consume(0, k0, jnp.minimum(k0 + 2, UNITS_PER_W - 2))
            consume(1, k0 + 1, jnp.minimum(k0 + 3, UNITS_PER_W - 1))
            return carry

        lax.fori_loop(0, UNITS_PER_W // 2, pair_body, 0)

        for par in (0, 1):
            pltpu.make_async_copy(
                stage_view(par), out_hbm.at[0, :, pl.ds(0, 2)],
                sws[par]).wait()
            pltpu.make_async_copy(
                lut_hbm.at[idx_slice(0)], gbufs[par], sgs[par]).wait()

    return lookup


def kernel(x, lut):
    xt = x.T.reshape(-1)
    op = _make_lookup()(lut, xt)
    return op.transpose(2, 4, 0, 1, 3).reshape(BATCH, SEQ, D_MODEL)
